# baseline (speedup 1.0000x reference)
import numpy as np
import jax
import jax.numpy as jnp
from jax.experimental import pallas as pl
from jax.experimental.pallas import tpu as pltpu

_N = 1024
_HALF = _N // 2
_NUM_WL = 31
_W = _N * _NUM_WL
_ROW_BLOCK = 32
_SI = np.float32(4e-06)
_RADIUS = np.float32(4e-06 * _N / 2.0)
_COORD_SCALE = np.float32(_SI / _RADIUS)


_TWO_OVER_PI = np.float32(0.6366197723675814)
_PIO2_HI = np.float32(1.5703125)
_PIO2_MD = np.float32(0.0004837512969970703)
_PIO2_LO = np.float32(7.549790126404332e-08)
_S1 = np.float32(-1.6666654611e-01)
_S2 = np.float32(8.3321608736e-03)
_S3 = np.float32(-1.9515295891e-04)
_C1 = np.float32(-0.5)
_C2 = np.float32(4.166664568298827e-02)
_C3 = np.float32(-1.388731625493765e-03)
_C4 = np.float32(2.443315711809948e-05)


def _sincos(theta):
    nf = jnp.floor(theta * _TWO_OVER_PI + jnp.float32(0.5))
    r = theta - nf * _PIO2_HI
    r = r - nf * _PIO2_MD
    r = r - nf * _PIO2_LO
    r2 = r * r
    sp = r + (r * r2) * (_S1 + r2 * (_S2 + r2 * _S3))
    cp = jnp.float32(1.0) + r2 * (_C1 + r2 * (_C2 + r2 * (_C3 + r2 * _C4)))
    n = nf.astype(jnp.int32)
    swap = (n & 1) == 1
    sin_v = jnp.where(swap, cp, sp)
    cos_v = jnp.where(swap, sp, cp)
    sin_v = jnp.where((n & 2) == 2, -sin_v, sin_v)
    cos_v = jnp.where(((n + 1) & 2) == 2, -cos_v, cos_v)
    return sin_v, cos_v


def _phase_mod_kernel(coef_ref, x2_ref, k_ref, in_ref, re_ref, im_ref):
    i = pl.program_id(0)
    row = jax.lax.broadcasted_iota(jnp.int32, (_ROW_BLOCK, 1), 0).astype(
        jnp.float32
    ) + (jnp.float32(i * _ROW_BLOCK) - np.float32(_HALF))
    r2 = x2_ref[...] + row * row
    r = jnp.sqrt(r2)
    idxf = jnp.clip(jnp.ceil(r) - 1.0, 0.0, np.float32(_HALF - 1))
    c = idxf * _COORD_SCALE
    t = c * c
    f = coef_ref[0] + t * (
        coef_ref[1] + t * (coef_ref[2] + t * (coef_ref[3] + t * coef_ref[4]))
    )
    theta = k_ref[...] * f
    sin_t, cos_t = _sincos(theta)
    inp = in_ref[...]
    mask = r2 <= np.float32(_HALF * _HALF)
    zero = jnp.float32(0.0)
    re_ref[...] = jnp.where(mask, inp * cos_t, zero)
    im_ref[...] = jnp.where(mask, inp * sin_t, zero)


def kernel(Input_field, coefficient, wavelength, step, writer):
    x = Input_field.reshape(_N, _W)
    coef = (coefficient * jnp.max(wavelength)).astype(jnp.float32)
    k = (2.0 * np.float32(np.pi)) / wavelength.astype(jnp.float32)
    k_flat = jnp.tile(k, _N).reshape(1, _W)
    u = jnp.arange(_N, dtype=jnp.float32) - np.float32(_HALF)
    x2_flat = jnp.repeat(u * u, _NUM_WL).reshape(1, _W)

    re, im = pl.pallas_call(
        _phase_mod_kernel,
        grid=(_N // _ROW_BLOCK,),
        in_specs=[
            pl.BlockSpec(memory_space=pltpu.SMEM),
            pl.BlockSpec((1, _W), lambda i: (0, 0)),
            pl.BlockSpec((1, _W), lambda i: (0, 0)),
            pl.BlockSpec((_ROW_BLOCK, _W), lambda i: (i, 0)),
        ],
        out_specs=[
            pl.BlockSpec((_ROW_BLOCK, _W), lambda i: (i, 0)),
            pl.BlockSpec((_ROW_BLOCK, _W), lambda i: (i, 0)),
        ],
        out_shape=[
            jax.ShapeDtypeStruct((_N, _W), jnp.float32),
            jax.ShapeDtypeStruct((_N, _W), jnp.float32),
        ],
        compiler_params=pltpu.CompilerParams(
            dimension_semantics=("parallel",),
        ),
    )(coef, x2_flat, k_flat, x)
    out = re.astype(jnp.complex64) + im.astype(jnp.complex64) * np.complex64(1j)
    return out.reshape(1, _N, _N, _NUM_WL)


# speedup vs baseline: 1.0233x; 1.0233x over previous
import numpy as np
import jax
import jax.numpy as jnp
from jax.experimental import pallas as pl
from jax.experimental.pallas import tpu as pltpu

_N = 1024
_HALF = _N // 2
_NUM_WL = 31
_W = _N * _NUM_WL
_ROW_BLOCK = 32
_SI = np.float32(4e-06)
_RADIUS = np.float32(4e-06 * _N / 2.0)
_COORD_SCALE = np.float32(_SI / _RADIUS)


_TWO_OVER_PI = np.float32(0.6366197723675814)
_PIO2_HI = np.float32(1.5703125)
_PIO2_MD = np.float32(0.0004837512969970703)
_PIO2_LO = np.float32(7.549790126404332e-08)
_S1 = np.float32(-1.6666654611e-01)
_S2 = np.float32(8.3321608736e-03)
_S3 = np.float32(-1.9515295891e-04)
_C1 = np.float32(-0.5)
_C2 = np.float32(4.166664568298827e-02)
_C3 = np.float32(-1.388731625493765e-03)
_C4 = np.float32(2.443315711809948e-05)


def _sincos(theta):
    nf = jnp.floor(theta * _TWO_OVER_PI + jnp.float32(0.5))
    r = theta - nf * _PIO2_HI
    r = r - nf * _PIO2_MD
    r = r - nf * _PIO2_LO
    r2 = r * r
    sp = r + (r * r2) * (_S1 + r2 * (_S2 + r2 * _S3))
    cp = jnp.float32(1.0) + r2 * (_C1 + r2 * (_C2 + r2 * (_C3 + r2 * _C4)))
    n = nf.astype(jnp.int32)
    swap = (n & 1) == 1
    sin_v = jnp.where(swap, cp, sp)
    cos_v = jnp.where(swap, sp, cp)
    sin_v = jnp.where((n & 2) == 2, -sin_v, sin_v)
    cos_v = jnp.where(((n + 1) & 2) == 2, -cos_v, cos_v)
    return sin_v, cos_v


def _phase_mod_kernel(coef_ref, x2_ref, k_ref, in_ref, re_ref, im_ref):
    i = pl.program_id(0)
    row = jax.lax.broadcasted_iota(jnp.int32, (_ROW_BLOCK, 1), 0).astype(
        jnp.float32
    ) + (jnp.float32(i * _ROW_BLOCK) - np.float32(_HALF))
    r2 = x2_ref[...] + row * row
    r = jnp.sqrt(r2)
    idxf = jnp.clip(jnp.ceil(r) - 1.0, 0.0, np.float32(_HALF - 1))
    c = idxf * _COORD_SCALE
    t = c * c
    f = coef_ref[0] + t * (
        coef_ref[1] + t * (coef_ref[2] + t * (coef_ref[3] + t * coef_ref[4]))
    )
    theta = k_ref[...] * f
    sin_t, cos_t = _sincos(theta)
    inp = in_ref[...]
    mask = r2 <= np.float32(_HALF * _HALF)
    zero = jnp.float32(0.0)
    re_ref[...] = jnp.where(mask, inp * cos_t, zero)
    im_ref[...] = jnp.where(mask, inp * sin_t, zero)


def kernel(Input_field, coefficient, wavelength, step, writer):
    x = Input_field.reshape(_N, _W)
    coef = (coefficient * jnp.max(wavelength)).astype(jnp.float32)
    k = (2.0 * np.float32(np.pi)) / wavelength.astype(jnp.float32)
    k_flat = jnp.tile(k, _N).reshape(1, _W)
    u = jnp.arange(_N, dtype=jnp.float32) - np.float32(_HALF)
    x2_flat = jnp.repeat(u * u, _NUM_WL).reshape(1, _W)

    re, im = pl.pallas_call(
        _phase_mod_kernel,
        grid=(_N // _ROW_BLOCK,),
        in_specs=[
            pl.BlockSpec(memory_space=pltpu.SMEM),
            pl.BlockSpec((1, _W), lambda i: (0, 0)),
            pl.BlockSpec((1, _W), lambda i: (0, 0)),
            pl.BlockSpec((_ROW_BLOCK, _W), lambda i: (i, 0)),
        ],
        out_specs=[
            pl.BlockSpec((_ROW_BLOCK, _W), lambda i: (i, 0)),
            pl.BlockSpec((_ROW_BLOCK, _W), lambda i: (i, 0)),
        ],
        out_shape=[
            jax.ShapeDtypeStruct((_N, _W), jnp.float32),
            jax.ShapeDtypeStruct((_N, _W), jnp.float32),
        ],
        compiler_params=pltpu.CompilerParams(
            dimension_semantics=("parallel",),
        ),
    )(coef, x2_flat, k_flat, x)
    vb = jax.lax.bitcast_convert_type(Input_field.reshape(-1)[0], jnp.uint32)
    one = ((vb & jnp.uint32(0)) | jnp.uint32(1)).astype(jnp.float32)
    out = jax.lax.complex(re * one, im * one)
    return out.reshape(1, _N, _N, _NUM_WL)
